# baseline (speedup 1.0000x reference)
import functools

import numpy as np
import jax
import jax.numpy as jnp
from jax import lax
from jax.experimental import pallas as pl
from jax.experimental.pallas import tpu as pltpu
from jax.experimental.pallas import tpu_sc as plsc

_NF = 26
_VOCAB = 100000
_DE = 32
_DM = 128
_B = 1024
_L = 50
_BL = _B * _L
_FAN_IN = _NF * _DE
_FAN_PAD = 896
_NTILE = 7
_NGRP = _BL // 8

_NC = 2
_NS = 16
_NW = _NC * _NS
_GRP_W = _NGRP // _NW
_IDX_W = _GRP_W * 8 * _NF
_ROWS_GRP = 8 * 4 * _NTILE
_ROWS_W = _GRP_W * _ROWS_GRP
_TOT_ROWS = _NGRP * _ROWS_GRP
_G = 128
_NG_W = _ROWS_W // _G
_CG = 5
_CHUNK = _CG * _G
_NCHUNK = _NG_W // _CG

_PERM_NP = np.zeros(_ROWS_GRP, dtype=np.int32)
_OFF_NP = np.zeros(_ROWS_GRP, dtype=np.int32)
for _j in range(_NTILE):
    for _r in range(8):
        for _p in range(4):
            _i = 4 * _j + _p
            _k = _j * 32 + _r * 4 + _p
            if _i < _NF:
                _PERM_NP[_k] = _r * _NF + _i
                _OFF_NP[_k] = 4 * _i * _VOCAB
            else:
                _PERM_NP[_k] = _r * _NF
                _OFF_NP[_k] = 0

_mesh = plsc.VectorSubcoreMesh(core_axis_name="c", subcore_axis_name="s")

_BV = 1024
_NBV = -(-_VOCAB // _BV)


def _tp_body(x_ref, o_ref):
    o_ref[0, :, 0:32] = x_ref[0].T


def _tp(tT):
    return pl.pallas_call(
        _tp_body,
        grid=(_NF, _NBV),
        in_specs=[pl.BlockSpec((1, 32, _BV), lambda i, v: (i, 0, v))],
        out_specs=pl.BlockSpec((1, _BV, 128), lambda i, v: (i, v, 0)),
        out_shape=jax.ShapeDtypeStruct((_NF, _VOCAB, 128), jnp.float32),
    )(tT)


def _gather_body(samp_hbm, table_hbm, perm_hbm, off_hbm, out_hbm,
                 samp_v, idxp_v, perm_v, off_v, rows_v, gsem):
    wid = lax.axis_index("s") * _NC + lax.axis_index("c")
    pltpu.sync_copy(perm_hbm, perm_v)
    pltpu.sync_copy(off_hbm, off_v)
    pltpu.sync_copy(samp_hbm.at[wid], samp_v)

    @pl.loop(0, _GRP_W)
    def _(g):
        sb = g * (8 * _NF)
        tb = g * _ROWS_GRP
        for s in range(_ROWS_GRP // 16):
            pv = perm_v[pl.ds(s * 16, 16)] + sb
            vals = plsc.load_gather(samp_v, [pv])
            idxp_v[pl.ds(tb + s * 16, 16)] = (
                vals * 4 + off_v[pl.ds(s * 16, 16)]
            )

    base = wid * _ROWS_W

    @pl.loop(0, _NCHUNK)
    def _(c):
        copies = []
        for g in range(_CG):
            copies.append(
                pltpu.async_copy(
                    table_hbm.at[idxp_v.at[pl.ds((c * _CG + g) * _G, _G)]],
                    rows_v.at[pl.ds(g * _G, _G)],
                    gsem,
                )
            )
        for cp in copies:
            cp.wait()
        pltpu.sync_copy(rows_v, out_hbm.at[pl.ds(base + c * _CHUNK, _CHUNK)])


def _sc_gather(samp_rs, tables_flat, perm, off):
    k = functools.partial(
        pl.kernel,
        mesh=_mesh,
        compiler_params=pltpu.CompilerParams(
            use_tc_tiling_on_sc=False, needs_layout_passes=False
        ),
        out_type=jax.ShapeDtypeStruct((_TOT_ROWS, _DE), jnp.float32),
        name="sc_gather",
        scratch_types=[
            pltpu.VMEM((_IDX_W,), jnp.int32),
            pltpu.VMEM((_ROWS_W,), jnp.int32),
            pltpu.VMEM((_ROWS_GRP,), jnp.int32),
            pltpu.VMEM((_ROWS_GRP,), jnp.int32),
            pltpu.VMEM((_CHUNK, _DE), jnp.float32),
            pltpu.SemaphoreType.DMA,
        ],
    )(_gather_body)
    return k(samp_rs, tables_flat, perm, off)


_BG = 256


def _mm_body(x_ref, w_ref, b_ref, o_ref):
    acc = jnp.broadcast_to(b_ref[...], (_BG * 8, _DM))
    for j in range(_NTILE):
        xj = x_ref[:, j].reshape(_BG * 8, _DM)
        acc = acc + jnp.dot(xj, w_ref[j], preferred_element_type=jnp.float32)
    o_ref[...] = acc


def _mm(x4d, w4, b2):
    return pl.pallas_call(
        _mm_body,
        grid=(_NGRP // _BG,),
        in_specs=[
            pl.BlockSpec((_BG, _NTILE, 8, _DM), lambda i: (i, 0, 0, 0)),
            pl.BlockSpec((_NTILE, _DM, _DM), lambda i: (0, 0, 0)),
            pl.BlockSpec((1, _DM), lambda i: (0, 0)),
        ],
        out_specs=pl.BlockSpec((_BG * 8, _DM), lambda i: (i, 0)),
        out_shape=jax.ShapeDtypeStruct((_BL, _DM), jnp.float32),
    )(x4d, w4, b2)


def kernel(sample, tables, W, b):
    samp_rs = sample.reshape(_NW, _IDX_W)
    tpad = _tp(jnp.swapaxes(tables, 1, 2))
    tables_flat = tpad.reshape(4 * _NF * _VOCAB, _DE)
    perm = jnp.asarray(_PERM_NP)
    off = jnp.asarray(_OFF_NP)
    gathered = _sc_gather(samp_rs, tables_flat, perm, off)
    x4d = gathered.reshape(_NGRP, _NTILE, 8, _DM)
    w4 = (
        jnp.zeros((_FAN_PAD, _DM), jnp.float32)
        .at[:_FAN_IN]
        .set(W)
        .reshape(_NTILE, _DM, _DM)
    )
    out = _mm(x4d, w4, b.reshape(1, _DM))
    return out.reshape(_B, _L, _DM)


# speedup vs baseline: 1.4750x; 1.4750x over previous
import functools

import numpy as np
import jax
import jax.numpy as jnp
from jax import lax
from jax.experimental import pallas as pl
from jax.experimental.pallas import tpu as pltpu
from jax.experimental.pallas import tpu_sc as plsc

_NF = 26
_VOCAB = 100000
_DE = 32
_DM = 128
_B = 1024
_L = 50
_BL = _B * _L
_FAN_IN = _NF * _DE
_FAN_PAD = 896
_NTILE = 7
_NGRP = _BL // 8

_NC = 2
_NS = 16
_NW = _NC * _NS
_GRP_W = _NGRP // _NW
_IDX_W = _GRP_W * 8 * _NF
_ROWS_GRP = 8 * 4 * _NTILE
_ROWS_W = _GRP_W * _ROWS_GRP
_TOT_ROWS = _NGRP * _ROWS_GRP
_G = 128
_NG_W = _ROWS_W // _G
_CG = 5
_CHUNK = _CG * _G
_NCHUNK = _NG_W // _CG

_PERM_NP = np.zeros(_ROWS_GRP, dtype=np.int32)
_OFF_NP = np.zeros(_ROWS_GRP, dtype=np.int32)
for _j in range(_NTILE):
    for _r in range(8):
        for _p in range(4):
            _i = 4 * _j + _p
            _k = _j * 32 + _r * 4 + _p
            if _i < _NF:
                _PERM_NP[_k] = _r * _NF + _i
                _OFF_NP[_k] = 4 * _i * _VOCAB
            else:
                _PERM_NP[_k] = _r * _NF
                _OFF_NP[_k] = 0

_mesh = plsc.VectorSubcoreMesh(core_axis_name="c", subcore_axis_name="s")

_BV = 1024
_NBV = -(-_VOCAB // _BV)


def _tp_body(x_ref, o_ref):
    o_ref[0, :, 0:32] = x_ref[0].T


def _tp(tT):
    return pl.pallas_call(
        _tp_body,
        grid=(_NF, _NBV),
        in_specs=[pl.BlockSpec((1, 32, _BV), lambda i, v: (i, 0, v))],
        out_specs=pl.BlockSpec((1, _BV, 128), lambda i, v: (i, v, 0)),
        out_shape=jax.ShapeDtypeStruct((_NF, _VOCAB, 128), jnp.float32),
    )(tT)


def _gather_body(samp_hbm, table_hbm, perm_hbm, off_hbm, out_hbm,
                 samp_v, idxp_v, perm_v, off_v, rows_v, gsem):
    wid = lax.axis_index("s") * _NC + lax.axis_index("c")
    pltpu.sync_copy(perm_hbm, perm_v)
    pltpu.sync_copy(off_hbm, off_v)
    pltpu.sync_copy(samp_hbm.at[wid], samp_v)

    @pl.loop(0, _GRP_W)
    def _(g):
        sb = g * (8 * _NF)
        tb = g * _ROWS_GRP
        for s in range(_ROWS_GRP // 16):
            pv = perm_v[pl.ds(s * 16, 16)] + sb
            vals = plsc.load_gather(samp_v, [pv])
            idxp_v[pl.ds(tb + s * 16, 16)] = (
                vals * 4 + off_v[pl.ds(s * 16, 16)]
            )

    base = wid * _ROWS_W

    @pl.loop(0, _NCHUNK)
    def _(c):
        copies = []
        for g in range(_CG):
            copies.append(
                pltpu.async_copy(
                    table_hbm.at[idxp_v.at[pl.ds((c * _CG + g) * _G, _G)]],
                    rows_v.at[pl.ds(g * _G, _G)],
                    gsem,
                )
            )
        for cp in copies:
            cp.wait()
        pltpu.sync_copy(rows_v, out_hbm.at[pl.ds(base + c * _CHUNK, _CHUNK)])


def _sc_gather(samp_rs, tables_flat, perm, off):
    k = functools.partial(
        pl.kernel,
        mesh=_mesh,
        compiler_params=pltpu.CompilerParams(
            use_tc_tiling_on_sc=False, needs_layout_passes=False
        ),
        out_type=jax.ShapeDtypeStruct((_TOT_ROWS, _DE), jnp.float32),
        name="sc_gather",
        scratch_types=[
            pltpu.VMEM((_IDX_W,), jnp.int32),
            pltpu.VMEM((_ROWS_W,), jnp.int32),
            pltpu.VMEM((_ROWS_GRP,), jnp.int32),
            pltpu.VMEM((_ROWS_GRP,), jnp.int32),
            pltpu.VMEM((_CHUNK, _DE), jnp.float32),
            pltpu.SemaphoreType.DMA,
        ],
    )(_gather_body)
    return k(samp_rs, tables_flat, perm, off)


_BG = 256


def _mm_body(x_ref, w_ref, b_ref, o_ref):
    acc = jnp.broadcast_to(b_ref[...], (_BG * 8, _DM))
    for j in range(_NTILE):
        xj = x_ref[:, j].reshape(_BG * 8, _DM)
        acc = acc + jnp.dot(xj, w_ref[j], preferred_element_type=jnp.float32)
    o_ref[...] = acc


def _mm(x4d, w4, b2):
    return pl.pallas_call(
        _mm_body,
        grid=(_NGRP // _BG,),
        in_specs=[
            pl.BlockSpec((_BG, _NTILE, 8, _DM), lambda i: (i, 0, 0, 0)),
            pl.BlockSpec((_NTILE, _DM, _DM), lambda i: (0, 0, 0)),
            pl.BlockSpec((1, _DM), lambda i: (0, 0)),
        ],
        out_specs=pl.BlockSpec((_BG * 8, _DM), lambda i: (i, 0)),
        out_shape=jax.ShapeDtypeStruct((_BL, _DM), jnp.float32),
    )(x4d, w4, b2)


def kernel(sample, tables, W, b):
    samp_rs = sample.reshape(_NW, _IDX_W)
    tpad = jnp.pad(tables, ((0, 0), (0, 0), (0, 128 - _DE)))
    tables_flat = tpad.reshape(4 * _NF * _VOCAB, _DE)
    perm = jnp.asarray(_PERM_NP)
    off = jnp.asarray(_OFF_NP)
    gathered = _sc_gather(samp_rs, tables_flat, perm, off)
    x4d = gathered.reshape(_NGRP, _NTILE, 8, _DM)
    w4 = (
        jnp.zeros((_FAN_PAD, _DM), jnp.float32)
        .at[:_FAN_IN]
        .set(W)
        .reshape(_NTILE, _DM, _DM)
    )
    out = _mm(x4d, w4, b.reshape(1, _DM))
    return out.reshape(_B, _L, _DM)
